# baseline (speedup 1.0000x reference)
import functools

import jax
import jax.numpy as jnp
from jax import lax
from jax.experimental import pallas as pl
from jax.experimental.pallas import tpu as pltpu
from jax.experimental.pallas import tpu_sc as plsc

B = 16
S = 512
D = 768
BS = B * S
EPS = 1e-12

_info = plsc.get_sparse_core_info()
NC = _info.num_cores
NS = _info.num_subcores
NW = NC * NS

K = 4
SLICE = BS // K
TOK_PER_W = SLICE // NW
CH = 32
NCH = TOK_PER_W // CH
NBUF = 2

TBLK = 512
NBLK = SLICE // TBLK


@functools.partial(
    pl.kernel,
    out_type=jax.ShapeDtypeStruct((SLICE, D), jnp.float32),
    mesh=plsc.VectorSubcoreMesh(core_axis_name="c", subcore_axis_name="s"),
    compiler_params=pltpu.CompilerParams(needs_layout_passes=False),
    scratch_types=(
        [pltpu.VMEM((NCH, CH), jnp.int32)]
        + [pltpu.VMEM((CH, D), jnp.float32) for _ in range(NBUF)]
        + [pltpu.SemaphoreType.DMA for _ in range(2 * NBUF)]
    ),
)
def _gather_sc(ids_hbm, tok_hbm, out_hbm, idx_v, *rest):
    bufs = list(rest[:NBUF])
    gsem = list(rest[NBUF:2 * NBUF])
    ssem = list(rest[2 * NBUF:])

    w = lax.axis_index("s") * NC + lax.axis_index("c")
    base = w * TOK_PER_W

    pltpu.sync_copy(ids_hbm.at[w], idx_v)

    def start_gather(c):
        return pltpu.async_copy(
            tok_hbm.at[idx_v.at[c]], bufs[c % NBUF], gsem[c % NBUF])

    def start_out(c):
        return pltpu.async_copy(
            bufs[c % NBUF], out_hbm.at[pl.ds(base + c * CH, CH)],
            ssem[c % NBUF])

    ghandles = {}
    shandles = {}
    for c in range(min(NBUF, NCH)):
        ghandles[c] = start_gather(c)
    for c in range(NCH):
        ghandles[c].wait()
        shandles[c] = start_out(c)
        n = c + NBUF
        if n < NCH:
            shandles[n - NBUF].wait()
            ghandles[n] = start_gather(n)
    for c in range(max(0, NCH - NBUF), NCH):
        shandles[c].wait()


def _ln_math(x, pos, g, b):
    e = x + pos
    mean = jnp.mean(e, axis=1, keepdims=True)
    var = jnp.mean(jnp.square(e - mean), axis=1, keepdims=True)
    return (e - mean) * lax.rsqrt(var + EPS) * g + b


def _ln_head_body(x_ref, pos_ref, g_ref, b_ref, o_ref):
    o_ref[...] = _ln_math(x_ref[...], pos_ref[...], g_ref[...], b_ref[...])


def _ln_tail_body(prev_ref, x_ref, pos_ref, g_ref, b_ref, o_ref):
    del prev_ref
    o_ref[...] = _ln_math(x_ref[...], pos_ref[...], g_ref[...], b_ref[...])


_DATA_SPECS = [
    pl.BlockSpec((TBLK, D), lambda i: (i, 0)),
    pl.BlockSpec((S, D), lambda i: (0, 0)),
    pl.BlockSpec((1, D), lambda i: (0, 0)),
    pl.BlockSpec((1, D), lambda i: (0, 0)),
]

_ln_head = pl.pallas_call(
    _ln_head_body,
    grid=(NBLK,),
    in_specs=_DATA_SPECS,
    out_specs=pl.BlockSpec((TBLK, D), lambda i: (i, 0)),
    out_shape=jax.ShapeDtypeStruct((BS, D), jnp.float32),
)

_ln_tails = [
    pl.pallas_call(
        _ln_tail_body,
        grid=(NBLK,),
        in_specs=[pl.BlockSpec(memory_space=pl.ANY)] + _DATA_SPECS,
        out_specs=pl.BlockSpec(
            (TBLK, D), functools.partial(lambda k, i: (k * NBLK + i, 0), k)),
        out_shape=jax.ShapeDtypeStruct((BS, D), jnp.float32),
        input_output_aliases={0: 0},
    )
    for k in range(1, K)
]


def kernel(input_ids, token_table, pos_table, ln_gamma, ln_beta):
    ids_g = input_ids.reshape(K, NW, NCH, CH)
    g2 = ln_gamma.reshape(1, D)
    b2 = ln_beta.reshape(1, D)
    embs = [_gather_sc(ids_g[k], token_table) for k in range(K)]
    buf = _ln_head(embs[0], pos_table, g2, b2)
    for k in range(1, K):
        buf = _ln_tails[k - 1](buf, embs[k], pos_table, g2, b2)
    return buf.reshape(B, S, D)
